# Initial kernel scaffold; baseline (speedup 1.0000x reference)
import jax
import jax.numpy as jnp
from jax.experimental import pallas as pl


def kernel(x, edge_index, edge_w, state_v, state_c, action, W1x, W1w, W1f, b1f, wvec1, W2x, W2w, W2f, b2f, wvec2, W5, b5, W6, b6, W7, b7, W8, b8, W9, b9):
    raise NotImplementedError("write your pallas kernel here")


# speedup vs baseline: 5.4642x; 5.4642x over previous
import functools

import jax
import jax.numpy as jnp
from jax import lax
from jax.experimental import pallas as pl
from jax.experimental.pallas import tpu as pltpu
from jax.experimental.pallas import tpu_sc as plsc

N = 50000
E = 800000
B = 50
NPER = 1000
EF = 64

NC = 2
NS = 16
CH = 128
NCHUNK = E // CH

HALF = N // NC
ACCB_ROWS = 25600
DUMMY = HALF
ZB = 80
ZA = 200

_mesh = plsc.VectorSubcoreMesh(
    core_axis_name="c", subcore_axis_name="s", num_cores=NC, num_subcores=NS)


@functools.partial(
    pl.kernel,
    out_type=pltpu.MemorySpace.HBM((NC, N, 16), jnp.float32),
    mesh=_mesh,
    scratch_types=[
        pltpu.VMEM_SHARED((N, 16), jnp.float32),
        pltpu.VMEM((CH, 16), jnp.float32),
        pltpu.VMEM((CH,), jnp.int32),
        pltpu.VMEM((CH,), jnp.int32),
        pltpu.VMEM((CH,), jnp.float32),
        pltpu.VMEM((ZA, 16), jnp.float32),
        pltpu.SemaphoreType.DMA,
    ],
    compiler_params=pltpu.CompilerParams(needs_layout_passes=False, use_tc_tiling_on_sc=False),
)
def _pass_a(ei_ref, w_ref, xtab_ref, zeros_ref, out_ref,
            acc, pay, srcb, dstb, wb, zbuf, sem):
    cid = lax.axis_index("c")
    sid = lax.axis_index("s")
    wid = cid * NS + sid

    nz = 15 + jnp.where(sid < 10, 1, 0)
    pltpu.sync_copy(zeros_ref, zbuf)

    def zbody(k, _):
        pltpu.sync_copy(zbuf, acc.at[pl.ds((sid + NS * k) * ZA, ZA)])
        return 0

    lax.fori_loop(0, nz, zbody, 0)
    plsc.subcore_barrier()

    niter = 195 + jnp.where(wid < 10, 1, 0)

    def body(i, _):
        base = (wid + 32 * i) * CH
        pltpu.sync_copy(ei_ref.at[0, pl.ds(base, CH)], srcb)
        pltpu.sync_copy(ei_ref.at[1, pl.ds(base, CH)], dstb)
        pltpu.sync_copy(w_ref.at[pl.ds(base, CH)], wb)
        pltpu.async_copy(xtab_ref.at[srcb], pay, sem).wait()
        rows0 = lax.iota(jnp.int32, 16)
        col2 = jnp.full((16,), 2, jnp.int32)
        for j in range(CH // 16):
            w16 = wb[pl.ds(j * 16, 16)]
            rows = rows0 + (j * 16)
            wp = jnp.maximum(w16, 0.0)
            wn = jnp.maximum(-w16, 0.0)
            plsc.store_scatter(pay, [rows, col2], wp)
            plsc.store_scatter(pay, [rows, col2 + 1], wn)
        pltpu.sync_copy(pay, acc.at[dstb], add=True)
        return 0

    lax.fori_loop(0, niter, body, 0)
    plsc.subcore_barrier()

    def obody(k, _):
        off = (sid + NS * k) * ZA
        pltpu.sync_copy(acc.at[pl.ds(off, ZA)], zbuf)
        pltpu.sync_copy(zbuf, out_ref.at[cid, pl.ds(off, ZA)])
        return 0

    lax.fori_loop(0, nz, obody, 0)


@functools.partial(
    pl.kernel,
    out_type=pltpu.MemorySpace.HBM((NC, ACCB_ROWS, EF), jnp.float32),
    mesh=_mesh,
    scratch_types=[
        pltpu.VMEM_SHARED((ACCB_ROWS, EF), jnp.float32),
        pltpu.VMEM((CH, EF), jnp.float32),
        pltpu.VMEM((CH,), jnp.int32),
        pltpu.VMEM((CH,), jnp.int32),
        pltpu.VMEM((CH,), jnp.int32),
        pltpu.VMEM((ZB, EF), jnp.float32),
        pltpu.SemaphoreType.DMA,
    ],
    compiler_params=pltpu.CompilerParams(needs_layout_passes=False, use_tc_tiling_on_sc=False),
)
def _pass_b(ei_ref, h_ref, zeros_ref, out_ref,
            acc, rows, srcb, dstb, idxb, zbuf, sem):
    cid = lax.axis_index("c")
    sid = lax.axis_index("s")
    nbase = cid * HALF

    pltpu.sync_copy(zeros_ref, zbuf)

    def zbody(k, _):
        pltpu.sync_copy(zbuf, acc.at[pl.ds((sid + NS * k) * ZB, ZB)])
        return 0

    lax.fori_loop(0, 20, zbody, 0)
    plsc.subcore_barrier()

    niter = 390 + jnp.where(sid < 10, 1, 0)

    def body(i, _):
        base = (sid + NS * i) * CH
        pltpu.sync_copy(ei_ref.at[0, pl.ds(base, CH)], srcb)
        pltpu.sync_copy(ei_ref.at[1, pl.ds(base, CH)], dstb)
        cp = pltpu.async_copy(h_ref.at[srcb], rows, sem)
        for j in range(CH // 16):
            d16 = dstb[pl.ds(j * 16, 16)]
            rel = d16 - nbase
            inr = (rel >= 0) & (rel < HALF)
            idxb[pl.ds(j * 16, 16)] = jnp.where(inr, rel, DUMMY)
        cp.wait()
        pltpu.sync_copy(rows, acc.at[idxb], add=True)
        return 0

    lax.fori_loop(0, niter, body, 0)
    plsc.subcore_barrier()

    def obody(k, _):
        off = (sid + NS * k) * ZB
        pltpu.sync_copy(acc.at[pl.ds(off, ZB)], zbuf)
        pltpu.sync_copy(zbuf, out_ref.at[cid, pl.ds(off, ZB)])
        return 0

    lax.fori_loop(0, 20, obody, 0)


RB = 2000


def _k1_body(outa_ref, x_ref, w1x_ref, w1f_ref, b1f_ref, wv1_ref, w1w_ref,
             h_ref):
    a1p = jnp.maximum(wv1_ref[...], 0.0) @ w1w_ref[...]
    a1n = jnp.maximum(-wv1_ref[...], 0.0) @ w1w_ref[...]
    agg = outa_ref[0] + outa_ref[1]
    sx = agg[:, 0:2]
    swp = agg[:, 2:3]
    swn = agg[:, 3:4]
    z = (jnp.dot(x_ref[...], w1x_ref[...], preferred_element_type=jnp.float32)
         + jnp.dot(sx, w1f_ref[...], preferred_element_type=jnp.float32)
         + b1f_ref[...] + swp * a1p + swn * a1n)
    h_ref[...] = jnp.maximum(z, 0.0)


def _run_k1(outa, x, W1x, W1f, b1f, wvec1, W1w):
    grid = N // RB
    return pl.pallas_call(
        _k1_body,
        grid=(grid,),
        in_specs=[
            pl.BlockSpec((NC, RB, 16), lambda i: (0, i, 0)),
            pl.BlockSpec((RB, 2), lambda i: (i, 0)),
            pl.BlockSpec((2, EF), lambda i: (0, 0)),
            pl.BlockSpec((2, EF), lambda i: (0, 0)),
            pl.BlockSpec((1, EF), lambda i: (0, 0)),
            pl.BlockSpec((1, EF), lambda i: (0, 0)),
            pl.BlockSpec((EF, EF), lambda i: (0, 0)),
        ],
        out_specs=pl.BlockSpec((RB, EF), lambda i: (i, 0)),
        out_shape=jax.ShapeDtypeStruct((N, EF), jnp.float32),
    )(outa, x, W1x, W1f, b1f, wvec1, W1w)


def _k2_body(sv_ref, sc_ref, ac_ref,
             x_ref, outa_ref, sh_ref,
             w2x_ref, w2f_ref, b2f_ref, wv2_ref, w2w_ref,
             w5_ref, b5_ref, w6_ref, b6_ref, w7_ref, b7_ref,
             w8_ref, b8_ref, w9_ref, b9_ref, q_ref):
    g = pl.program_id(0)
    a2p = jnp.maximum(wv2_ref[...], 0.0) @ w2w_ref[...]
    a2n = jnp.maximum(-wv2_ref[...], 0.0) @ w2w_ref[...]

    agg = outa_ref[0] + outa_ref[1]
    swp = agg[:, 2:3]
    swn = agg[:, 3:4]
    z = (jnp.dot(x_ref[...], w2x_ref[...], preferred_element_type=jnp.float32)
         + jnp.dot(sh_ref[...], w2f_ref[...],
                   preferred_element_type=jnp.float32)
         + b2f_ref[...] + swp * a2p + swn * a2n)
    h2 = jnp.maximum(z, 0.0)

    pooled = jnp.sum(h2, axis=0, keepdims=True)
    h1g = pooled @ w6_ref[...] + b6_ref[...]

    sv = sv_ref[g, 0]
    aggr = outa_ref[0, pl.ds(sv, 1), :] + outa_ref[1, pl.ds(sv, 1), :]
    zr = (jnp.dot(x_ref[pl.ds(sv, 1), :], w2x_ref[...],
                  preferred_element_type=jnp.float32)
          + jnp.dot(sh_ref[pl.ds(sv, 1), :], w2f_ref[...],
                    preferred_element_type=jnp.float32)
          + b2f_ref[...] + aggr[:, 2:3] * a2p + aggr[:, 3:4] * a2n)
    cur = jnp.maximum(zr, 0.0)
    h2g = (cur @ w7_ref[...] + b7_ref[...]
           + ac_ref[g, 0] * w8_ref[...] + b8_ref[...]
           + sc_ref[g, 0] * w9_ref[...] + b9_ref[...])

    hh = jnp.maximum(jnp.concatenate([h1g, h2g], axis=1), 0.0)
    q = jnp.dot(hh, w5_ref[...], preferred_element_type=jnp.float32) \
        + b5_ref[...]
    q_ref[...] = jnp.broadcast_to(q[:, None, :], (1, 8, 128))


def _run_k2(state_v, state_c, action, x, outa, sh,
            W2x, W2f, b2f, wvec2, W2w, W5, b5, W6, b6, W7, b7, W8, b8, W9, b9):
    wfull = lambda shape: pl.BlockSpec(shape, lambda g: tuple(0 for _ in shape))
    out = pl.pallas_call(
        _k2_body,
        grid=(B,),
        in_specs=[
            pl.BlockSpec(memory_space=pltpu.SMEM),
            pl.BlockSpec(memory_space=pltpu.SMEM),
            pl.BlockSpec(memory_space=pltpu.SMEM),
            pl.BlockSpec((NPER, 2), lambda g: (g, 0)),
            pl.BlockSpec((NC, NPER, 16), lambda g: (0, g, 0)),
            pl.BlockSpec((NPER, EF), lambda g: (g, 0)),
            wfull((2, EF)), wfull((EF, EF)), wfull((1, EF)),
            wfull((1, EF)), wfull((EF, EF)),
            wfull((2 * EF, 1)), wfull((1, 1)),
            wfull((EF, EF)), wfull((1, EF)),
            wfull((EF, EF)), wfull((1, EF)),
            wfull((1, EF)), wfull((1, EF)),
            wfull((1, EF)), wfull((1, EF)),
        ],
        out_specs=pl.BlockSpec((1, 8, 128), lambda g: (g, 0, 0)),
        out_shape=jax.ShapeDtypeStruct((B, 8, 128), jnp.float32),
    )(state_v, state_c, action, x, outa, sh,
      W2x, W2f, b2f, wvec2, W2w, W5, b5, W6, b6, W7, b7, W8, b8, W9, b9)
    return out[:, 0, :1]


def kernel(x, edge_index, edge_w, state_v, state_c, action,
           W1x, W1w, W1f, b1f, wvec1,
           W2x, W2w, W2f, b2f, wvec2,
           W5, b5, W6, b6, W7, b7, W8, b8, W9, b9):
    w = edge_w.reshape(E)

    zeros_a = jnp.zeros((ZA, 16), jnp.float32)
    zeros_b = jnp.zeros((ZB, EF), jnp.float32)

    xtab = jnp.pad(x, ((0, 0), (0, 14)))
    outa = _pass_a(edge_index, w, xtab, zeros_a)

    r1 = lambda v: v.reshape(1, -1)
    h = _run_k1(outa, x, W1x, W1f, r1(b1f), r1(wvec1), W1w)

    outb = _pass_b(edge_index, h, zeros_b)
    sh = jnp.concatenate([outb[0, :HALF], outb[1, :HALF]], axis=0)

    sv = state_v.reshape(B, 1)
    return _run_k2(sv, state_c, action, x, outa, sh,
                   W2x, W2f, r1(b2f), r1(wvec2), W2w,
                   W5, b5.reshape(1, 1), W6, r1(b6), W7, r1(b7),
                   r1(W8[0]), r1(b8), r1(W9[0]), r1(b9))


# speedup vs baseline: 12.4640x; 2.2811x over previous
import functools

import jax
import jax.numpy as jnp
from jax import lax
from jax.experimental import pallas as pl
from jax.experimental.pallas import tpu as pltpu
from jax.experimental.pallas import tpu_sc as plsc

N = 50000
E = 800000
B = 50
NPER = 1000
EF = 64
HF = EF // 2

NC = 2
NS = 16
CH = 128
K = 4

QA = 196
QB = 392
PADC = 6272
EPAD = PADC * CH
NROWS = N + 8
ZA = 200

_mesh = plsc.VectorSubcoreMesh(
    core_axis_name="c", subcore_axis_name="s", num_cores=NC, num_subcores=NS)

_sc_params = pltpu.CompilerParams(
    needs_layout_passes=False, use_tc_tiling_on_sc=False)


@functools.partial(
    pl.kernel,
    out_type=pltpu.MemorySpace.HBM((NC, N, 16), jnp.float32),
    mesh=_mesh,
    scratch_types=[
        pltpu.VMEM_SHARED((NROWS, 16), jnp.float32),
        [pltpu.VMEM((CH, 16), jnp.float32) for _ in range(K)],
        pltpu.VMEM((K * CH,), jnp.int32),
        pltpu.VMEM((K, CH), jnp.int32),
        pltpu.VMEM((K * CH,), jnp.float32),
        pltpu.VMEM((ZA, 16), jnp.float32),
        pltpu.SemaphoreType.DMA,
        pltpu.SemaphoreType.DMA,
    ],
    compiler_params=_sc_params,
)
def _pass_a(ei_ref, w_ref, xtab_ref, zeros_ref, out_ref,
            acc, pays, srcb, dstb, wb, zbuf, semi, semg):
    cid = lax.axis_index("c")
    sid = lax.axis_index("s")
    wid = cid * NS + sid

    nz = 15 + jnp.where(sid < 10, 1, 0)
    pltpu.sync_copy(zeros_ref, zbuf)

    def zbody(k, _):
        pltpu.sync_copy(zbuf, acc.at[pl.ds((sid + NS * k) * ZA, ZA)])
        return 0

    lax.fori_loop(0, nz, zbody, 0)
    plsc.subcore_barrier()

    rows0 = lax.iota(jnp.int32, 16)
    col2 = jnp.full((16,), 2, jnp.int32)

    def body(i, _):
        base = (wid * QA + K * i) * CH
        cpi = [pltpu.async_copy(ei_ref.at[0, pl.ds(base, K * CH)], srcb, semi),
               pltpu.async_copy(w_ref.at[pl.ds(base, K * CH)], wb, semi)]
        cpi += [pltpu.async_copy(ei_ref.at[1, pl.ds(base + k * CH, CH)],
                                 dstb.at[k], semi) for k in range(K)]
        for c in cpi:
            c.wait()
        gs = [pltpu.async_copy(
            xtab_ref.at[srcb.at[pl.ds(k * CH, CH)]], pays[k], semg)
            for k in range(K)]
        for k in range(K):
            gs[k].wait()
            for j in range(CH // 16):
                w16 = wb[pl.ds(k * CH + j * 16, 16)]
                rr = rows0 + (j * 16)
                plsc.store_scatter(pays[k], [rr, col2], jnp.maximum(w16, 0.0))
                plsc.store_scatter(pays[k], [rr, col2 + 1],
                                   jnp.maximum(-w16, 0.0))
            pltpu.sync_copy(pays[k], acc.at[dstb.at[k]], add=True)
        return 0

    lax.fori_loop(0, QA // K, body, 0)
    plsc.subcore_barrier()

    def obody(k, _):
        off = (sid + NS * k) * ZA
        pltpu.sync_copy(acc.at[pl.ds(off, ZA)], zbuf)
        pltpu.sync_copy(zbuf, out_ref.at[cid, pl.ds(off, ZA)])
        return 0

    lax.fori_loop(0, nz, obody, 0)


@functools.partial(
    pl.kernel,
    out_type=pltpu.MemorySpace.HBM((NC, N, HF), jnp.float32),
    mesh=_mesh,
    scratch_types=[
        pltpu.VMEM_SHARED((NROWS, HF), jnp.float32),
        [pltpu.VMEM((CH, HF), jnp.float32) for _ in range(K)],
        pltpu.VMEM((K * CH,), jnp.int32),
        pltpu.VMEM((K, CH), jnp.int32),
        pltpu.VMEM((ZA, HF), jnp.float32),
        pltpu.SemaphoreType.DMA,
        pltpu.SemaphoreType.DMA,
    ],
    compiler_params=_sc_params,
)
def _pass_b(ei_ref, h_ref, zeros_ref, out_ref,
            acc, rows, srcb, dstb, zbuf, semi, semg):
    cid = lax.axis_index("c")
    sid = lax.axis_index("s")

    nz = 15 + jnp.where(sid < 10, 1, 0)
    pltpu.sync_copy(zeros_ref, zbuf)

    def zbody(k, _):
        pltpu.sync_copy(zbuf, acc.at[pl.ds((sid + NS * k) * ZA, ZA)])
        return 0

    lax.fori_loop(0, nz, zbody, 0)
    plsc.subcore_barrier()

    def body(i, _):
        base = (sid * QB + K * i) * CH
        cpi = [pltpu.async_copy(ei_ref.at[0, pl.ds(base, K * CH)], srcb, semi)]
        cpi += [pltpu.async_copy(ei_ref.at[1, pl.ds(base + k * CH, CH)],
                                 dstb.at[k], semi) for k in range(K)]
        for c in cpi:
            c.wait()
        gs = [pltpu.async_copy(
            h_ref.at[cid].at[srcb.at[pl.ds(k * CH, CH)]], rows[k], semg)
            for k in range(K)]
        for k in range(K):
            gs[k].wait()
            pltpu.sync_copy(rows[k], acc.at[dstb.at[k]], add=True)
        return 0

    lax.fori_loop(0, QB // K, body, 0)
    plsc.subcore_barrier()

    def obody(k, _):
        off = (sid + NS * k) * ZA
        pltpu.sync_copy(acc.at[pl.ds(off, ZA)], zbuf)
        pltpu.sync_copy(zbuf, out_ref.at[cid, pl.ds(off, ZA)])
        return 0

    lax.fori_loop(0, nz, obody, 0)


RB = 2000


def _k1_body(outa_ref, x_ref, w1x_ref, w1f_ref, b1f_ref, wv1_ref, w1w_ref,
             h_ref):
    a1p = jnp.maximum(wv1_ref[...], 0.0) @ w1w_ref[...]
    a1n = jnp.maximum(-wv1_ref[...], 0.0) @ w1w_ref[...]
    agg = outa_ref[0] + outa_ref[1]
    sx = agg[:, 0:2]
    swp = agg[:, 2:3]
    swn = agg[:, 3:4]
    z = (jnp.dot(x_ref[...], w1x_ref[...], preferred_element_type=jnp.float32)
         + jnp.dot(sx, w1f_ref[...], preferred_element_type=jnp.float32)
         + b1f_ref[...] + swp * a1p + swn * a1n)
    h = jnp.maximum(z, 0.0)
    h_ref[0] = h[:, :HF]
    h_ref[1] = h[:, HF:]


def _run_k1(outa, x, W1x, W1f, b1f, wvec1, W1w):
    grid = N // RB
    return pl.pallas_call(
        _k1_body,
        grid=(grid,),
        in_specs=[
            pl.BlockSpec((NC, RB, 16), lambda i: (0, i, 0)),
            pl.BlockSpec((RB, 2), lambda i: (i, 0)),
            pl.BlockSpec((2, EF), lambda i: (0, 0)),
            pl.BlockSpec((2, EF), lambda i: (0, 0)),
            pl.BlockSpec((1, EF), lambda i: (0, 0)),
            pl.BlockSpec((1, EF), lambda i: (0, 0)),
            pl.BlockSpec((EF, EF), lambda i: (0, 0)),
        ],
        out_specs=pl.BlockSpec((2, RB, HF), lambda i: (0, i, 0)),
        out_shape=jax.ShapeDtypeStruct((2, N, HF), jnp.float32),
    )(outa, x, W1x, W1f, b1f, wvec1, W1w)


def _k2_body(sv_ref, sc_ref, ac_ref,
             x_ref, outa_ref, sh_ref,
             w2x_ref, w2f_ref, b2f_ref, wv2_ref, w2w_ref,
             w5_ref, b5_ref, w6_ref, b6_ref, w7_ref, b7_ref,
             w8_ref, b8_ref, w9_ref, b9_ref, q_ref):
    g = pl.program_id(0)
    a2p = jnp.maximum(wv2_ref[...], 0.0) @ w2w_ref[...]
    a2n = jnp.maximum(-wv2_ref[...], 0.0) @ w2w_ref[...]

    agg = outa_ref[0] + outa_ref[1]
    swp = agg[:, 2:3]
    swn = agg[:, 3:4]
    sh = jnp.concatenate([sh_ref[0], sh_ref[1]], axis=1)
    z = (jnp.dot(x_ref[...], w2x_ref[...], preferred_element_type=jnp.float32)
         + jnp.dot(sh, w2f_ref[...], preferred_element_type=jnp.float32)
         + b2f_ref[...] + swp * a2p + swn * a2n)
    h2 = jnp.maximum(z, 0.0)

    pooled = jnp.sum(h2, axis=0, keepdims=True)
    h1g = pooled @ w6_ref[...] + b6_ref[...]

    sv = sv_ref[g, 0]
    aggr = outa_ref[0, pl.ds(sv, 1), :] + outa_ref[1, pl.ds(sv, 1), :]
    shr = jnp.concatenate([sh_ref[0, pl.ds(sv, 1), :],
                           sh_ref[1, pl.ds(sv, 1), :]], axis=1)
    zr = (jnp.dot(x_ref[pl.ds(sv, 1), :], w2x_ref[...],
                  preferred_element_type=jnp.float32)
          + jnp.dot(shr, w2f_ref[...], preferred_element_type=jnp.float32)
          + b2f_ref[...] + aggr[:, 2:3] * a2p + aggr[:, 3:4] * a2n)
    cur = jnp.maximum(zr, 0.0)
    h2g = (cur @ w7_ref[...] + b7_ref[...]
           + ac_ref[g, 0] * w8_ref[...] + b8_ref[...]
           + sc_ref[g, 0] * w9_ref[...] + b9_ref[...])

    hh = jnp.maximum(jnp.concatenate([h1g, h2g], axis=1), 0.0)
    q = jnp.dot(hh, w5_ref[...], preferred_element_type=jnp.float32) \
        + b5_ref[...]
    q_ref[...] = jnp.broadcast_to(q[:, None, :], (1, 8, 128))


def _run_k2(state_v, state_c, action, x, outa, sh,
            W2x, W2f, b2f, wvec2, W2w, W5, b5, W6, b6, W7, b7, W8, b8, W9, b9):
    wfull = lambda shape: pl.BlockSpec(shape, lambda g: tuple(0 for _ in shape))
    out = pl.pallas_call(
        _k2_body,
        grid=(B,),
        in_specs=[
            pl.BlockSpec(memory_space=pltpu.SMEM),
            pl.BlockSpec(memory_space=pltpu.SMEM),
            pl.BlockSpec(memory_space=pltpu.SMEM),
            pl.BlockSpec((NPER, 2), lambda g: (g, 0)),
            pl.BlockSpec((NC, NPER, 16), lambda g: (0, g, 0)),
            pl.BlockSpec((2, NPER, HF), lambda g: (0, g, 0)),
            wfull((2, EF)), wfull((EF, EF)), wfull((1, EF)),
            wfull((1, EF)), wfull((EF, EF)),
            wfull((2 * EF, 1)), wfull((1, 1)),
            wfull((EF, EF)), wfull((1, EF)),
            wfull((EF, EF)), wfull((1, EF)),
            wfull((1, EF)), wfull((1, EF)),
            wfull((1, EF)), wfull((1, EF)),
        ],
        out_specs=pl.BlockSpec((1, 8, 128), lambda g: (g, 0, 0)),
        out_shape=jax.ShapeDtypeStruct((B, 8, 128), jnp.float32),
    )(state_v, state_c, action, x, outa, sh,
      W2x, W2f, b2f, wvec2, W2w, W5, b5, W6, b6, W7, b7, W8, b8, W9, b9)
    return out[:, 0, :1]


def kernel(x, edge_index, edge_w, state_v, state_c, action,
           W1x, W1w, W1f, b1f, wvec1,
           W2x, W2w, W2f, b2f, wvec2,
           W5, b5, W6, b6, W7, b7, W8, b8, W9, b9):
    npad = EPAD - E
    pad_ei = jnp.stack([jnp.zeros((npad,), jnp.int32),
                        jnp.full((npad,), N, jnp.int32)])
    ei_pad = jnp.concatenate([edge_index, pad_ei], axis=1)
    w_pad = jnp.concatenate([edge_w.reshape(E),
                             jnp.zeros((npad,), jnp.float32)])
    xtab = jnp.pad(x, ((0, 0), (0, 14)))

    zeros_a = jnp.zeros((ZA, 16), jnp.float32)
    zeros_b = jnp.zeros((ZA, HF), jnp.float32)

    outa = _pass_a(ei_pad, w_pad, xtab, zeros_a)

    r1 = lambda v: v.reshape(1, -1)
    h = _run_k1(outa, x, W1x, W1f, r1(b1f), r1(wvec1), W1w)

    outb = _pass_b(ei_pad, h, zeros_b)

    sv = state_v.reshape(B, 1)
    return _run_k2(sv, state_c, action, x, outa, outb,
                   W2x, W2f, r1(b2f), r1(wvec2), W2w,
                   W5, b5.reshape(1, 1), W6, r1(b6), W7, r1(b7),
                   r1(W8[0]), r1(b8), r1(W9[0]), r1(b9))


# speedup vs baseline: 14.5528x; 1.1676x over previous
import functools

import jax
import jax.numpy as jnp
from jax import lax
from jax.experimental import pallas as pl
from jax.experimental.pallas import tpu as pltpu
from jax.experimental.pallas import tpu_sc as plsc

N = 50000
E = 800000
B = 50
NPER = 1000
EF = 64
HF = EF // 2

NC = 2
NS = 16
CH = 128
K = 5
SCH = K * CH
NSUP = E // SCH
ZA = 200

_mesh = plsc.VectorSubcoreMesh(
    core_axis_name="c", subcore_axis_name="s", num_cores=NC, num_subcores=NS)

_sc_params = pltpu.CompilerParams(
    needs_layout_passes=False, use_tc_tiling_on_sc=False)


@functools.partial(
    pl.kernel,
    out_type=pltpu.MemorySpace.HBM((NC, N, 16), jnp.float32),
    mesh=_mesh,
    scratch_types=[
        pltpu.VMEM_SHARED((N, 16), jnp.float32),
        [pltpu.VMEM((CH, 16), jnp.float32) for _ in range(K)],
        pltpu.VMEM((SCH,), jnp.int32),
        pltpu.VMEM((K, CH), jnp.int32),
        pltpu.VMEM((SCH,), jnp.float32),
        pltpu.VMEM((ZA, 16), jnp.float32),
        pltpu.SemaphoreType.DMA,
        pltpu.SemaphoreType.DMA,
    ],
    compiler_params=_sc_params,
)
def _pass_a(ei_ref, w_ref, xtab_ref, zeros_ref, out_ref,
            acc, pays, srcb, dstb, wb, zbuf, semi, semg):
    cid = lax.axis_index("c")
    sid = lax.axis_index("s")
    wid = cid * NS + sid

    nz = 15 + jnp.where(sid < 10, 1, 0)
    pltpu.sync_copy(zeros_ref, zbuf)

    def zbody(k, _):
        pltpu.sync_copy(zbuf, acc.at[pl.ds((sid + NS * k) * ZA, ZA)])
        return 0

    lax.fori_loop(0, nz, zbody, 0)
    plsc.subcore_barrier()

    rows0 = lax.iota(jnp.int32, 16)
    col2 = jnp.full((16,), 2, jnp.int32)
    niter = 39 + jnp.where(wid < 2, 1, 0)

    def body(i, _):
        base = (wid + 32 * i) * SCH
        cpi = [pltpu.async_copy(ei_ref.at[0, pl.ds(base, SCH)], srcb, semi),
               pltpu.async_copy(w_ref.at[pl.ds(base, SCH)], wb, semi)]
        cpi += [pltpu.async_copy(ei_ref.at[1, pl.ds(base + k * CH, CH)],
                                 dstb.at[k], semi) for k in range(K)]
        for c in cpi:
            c.wait()
        gs = [pltpu.async_copy(
            xtab_ref.at[srcb.at[pl.ds(k * CH, CH)]], pays[k], semg)
            for k in range(K)]
        for k in range(K):
            gs[k].wait()
            for j in range(CH // 16):
                w16 = wb[pl.ds(k * CH + j * 16, 16)]
                rr = rows0 + (j * 16)
                plsc.store_scatter(pays[k], [rr, col2], jnp.maximum(w16, 0.0))
                plsc.store_scatter(pays[k], [rr, col2 + 1],
                                   jnp.maximum(-w16, 0.0))
            pltpu.sync_copy(pays[k], acc.at[dstb.at[k]], add=True)
        return 0

    lax.fori_loop(0, niter, body, 0)
    plsc.subcore_barrier()

    def obody(k, _):
        off = (sid + NS * k) * ZA
        pltpu.sync_copy(acc.at[pl.ds(off, ZA)], zbuf)
        pltpu.sync_copy(zbuf, out_ref.at[cid, pl.ds(off, ZA)])
        return 0

    lax.fori_loop(0, nz, obody, 0)


@functools.partial(
    pl.kernel,
    out_type=pltpu.MemorySpace.HBM((NC, N, HF), jnp.float32),
    mesh=_mesh,
    scratch_types=[
        pltpu.VMEM_SHARED((N, HF), jnp.float32),
        [pltpu.VMEM((CH, HF), jnp.float32) for _ in range(K)],
        pltpu.VMEM((SCH,), jnp.int32),
        pltpu.VMEM((K, CH), jnp.int32),
        pltpu.VMEM((ZA, HF), jnp.float32),
        pltpu.SemaphoreType.DMA,
        pltpu.SemaphoreType.DMA,
    ],
    compiler_params=_sc_params,
)
def _pass_b(ei_ref, h_ref, zeros_ref, out_ref,
            acc, rows, srcb, dstb, zbuf, semi, semg):
    cid = lax.axis_index("c")
    sid = lax.axis_index("s")

    nz = 15 + jnp.where(sid < 10, 1, 0)
    pltpu.sync_copy(zeros_ref, zbuf)

    def zbody(k, _):
        pltpu.sync_copy(zbuf, acc.at[pl.ds((sid + NS * k) * ZA, ZA)])
        return 0

    lax.fori_loop(0, nz, zbody, 0)
    plsc.subcore_barrier()

    niter = 78 + jnp.where(sid < 2, 1, 0)

    def body(i, _):
        base = (sid + NS * i) * SCH
        cpi = [pltpu.async_copy(ei_ref.at[0, pl.ds(base, SCH)], srcb, semi)]
        cpi += [pltpu.async_copy(ei_ref.at[1, pl.ds(base + k * CH, CH)],
                                 dstb.at[k], semi) for k in range(K)]
        for c in cpi:
            c.wait()
        gs = [pltpu.async_copy(
            h_ref.at[cid].at[srcb.at[pl.ds(k * CH, CH)]], rows[k], semg)
            for k in range(K)]
        for k in range(K):
            gs[k].wait()
            pltpu.sync_copy(rows[k], acc.at[dstb.at[k]], add=True)
        return 0

    lax.fori_loop(0, niter, body, 0)
    plsc.subcore_barrier()

    def obody(k, _):
        off = (sid + NS * k) * ZA
        pltpu.sync_copy(acc.at[pl.ds(off, ZA)], zbuf)
        pltpu.sync_copy(zbuf, out_ref.at[cid, pl.ds(off, ZA)])
        return 0

    lax.fori_loop(0, nz, obody, 0)


RB = 5000


def _k1_body(outa_ref, x_ref, w1x_ref, w1f_ref, b1f_ref, wv1_ref, w1w_ref,
             h_ref):
    a1p = jnp.maximum(wv1_ref[...], 0.0) @ w1w_ref[...]
    a1n = jnp.maximum(-wv1_ref[...], 0.0) @ w1w_ref[...]
    agg = outa_ref[0] + outa_ref[1]
    sx = agg[:, 0:2]
    swp = agg[:, 2:3]
    swn = agg[:, 3:4]
    z = (jnp.dot(x_ref[...], w1x_ref[...], preferred_element_type=jnp.float32)
         + jnp.dot(sx, w1f_ref[...], preferred_element_type=jnp.float32)
         + b1f_ref[...] + swp * a1p + swn * a1n)
    h = jnp.maximum(z, 0.0)
    h_ref[0] = h[:, :HF]
    h_ref[1] = h[:, HF:]


def _run_k1(outa, x, W1x, W1f, b1f, wvec1, W1w):
    grid = N // RB
    return pl.pallas_call(
        _k1_body,
        grid=(grid,),
        in_specs=[
            pl.BlockSpec((NC, RB, 16), lambda i: (0, i, 0)),
            pl.BlockSpec((RB, 2), lambda i: (i, 0)),
            pl.BlockSpec((2, EF), lambda i: (0, 0)),
            pl.BlockSpec((2, EF), lambda i: (0, 0)),
            pl.BlockSpec((1, EF), lambda i: (0, 0)),
            pl.BlockSpec((1, EF), lambda i: (0, 0)),
            pl.BlockSpec((EF, EF), lambda i: (0, 0)),
        ],
        out_specs=pl.BlockSpec((2, RB, HF), lambda i: (0, i, 0)),
        out_shape=jax.ShapeDtypeStruct((2, N, HF), jnp.float32),
    )(outa, x, W1x, W1f, b1f, wvec1, W1w)


GPB = 5
RG = GPB * NPER
NGB = B // GPB


def _k2a_body(sv_ref, x_ref, outa_ref, sh_ref,
              w2x_ref, w2f_ref, b2f_ref, wv2_ref, w2w_ref,
              pool_ref, cur_ref):
    g = pl.program_id(0)
    a2p = jnp.maximum(wv2_ref[...], 0.0) @ w2w_ref[...]
    a2n = jnp.maximum(-wv2_ref[...], 0.0) @ w2w_ref[...]

    agg = outa_ref[0] + outa_ref[1]
    sh = jnp.concatenate([sh_ref[0], sh_ref[1]], axis=1)
    z = (jnp.dot(x_ref[...], w2x_ref[...], preferred_element_type=jnp.float32)
         + jnp.dot(sh, w2f_ref[...], preferred_element_type=jnp.float32)
         + b2f_ref[...] + agg[:, 2:3] * a2p + agg[:, 3:4] * a2n)
    h2 = jnp.maximum(z, 0.0)

    pools = [jnp.sum(h2[gg * NPER:(gg + 1) * NPER], axis=0, keepdims=True)
             for gg in range(GPB)]
    pool_ref[...] = jnp.concatenate(pools, axis=0)[None]

    curs = []
    for gg in range(GPB):
        sv = sv_ref[g * GPB + gg, 0] + gg * NPER
        aggr = outa_ref[0, pl.ds(sv, 1), :] + outa_ref[1, pl.ds(sv, 1), :]
        shr = jnp.concatenate([sh_ref[0, pl.ds(sv, 1), :],
                               sh_ref[1, pl.ds(sv, 1), :]], axis=1)
        zr = (jnp.dot(x_ref[pl.ds(sv, 1), :], w2x_ref[...],
                      preferred_element_type=jnp.float32)
              + jnp.dot(shr, w2f_ref[...], preferred_element_type=jnp.float32)
              + b2f_ref[...] + aggr[:, 2:3] * a2p + aggr[:, 3:4] * a2n)
        curs.append(jnp.maximum(zr, 0.0))
    cur_ref[...] = jnp.concatenate(curs, axis=0)[None]


def _run_k2a(state_v, x, outa, sh, W2x, W2f, b2f, wvec2, W2w):
    wfull = lambda shape: pl.BlockSpec(shape, lambda g: tuple(0 for _ in shape))
    return pl.pallas_call(
        _k2a_body,
        grid=(NGB,),
        in_specs=[
            pl.BlockSpec(memory_space=pltpu.SMEM),
            pl.BlockSpec((RG, 2), lambda g: (g, 0)),
            pl.BlockSpec((NC, RG, 16), lambda g: (0, g, 0)),
            pl.BlockSpec((2, RG, HF), lambda g: (0, g, 0)),
            wfull((2, EF)), wfull((EF, EF)), wfull((1, EF)),
            wfull((1, EF)), wfull((EF, EF)),
        ],
        out_specs=[pl.BlockSpec((1, GPB, EF), lambda g: (g, 0, 0)),
                   pl.BlockSpec((1, GPB, EF), lambda g: (g, 0, 0))],
        out_shape=[jax.ShapeDtypeStruct((NGB, GPB, EF), jnp.float32),
                   jax.ShapeDtypeStruct((NGB, GPB, EF), jnp.float32)],
    )(state_v, x, outa, sh, W2x, W2f, b2f, wvec2, W2w)


def _k2b_body(pool_ref, cur_ref, sc_ref, ac_ref,
              w5_ref, b5_ref, w6_ref, b6_ref, w7_ref, b7_ref,
              w8_ref, b8_ref, w9_ref, b9_ref, q_ref):
    pooled = pool_ref[...].reshape(B, EF)
    cur = cur_ref[...].reshape(B, EF)
    h1 = jnp.dot(pooled, w6_ref[...],
                 preferred_element_type=jnp.float32) + b6_ref[...]
    h2 = (jnp.dot(cur, w7_ref[...], preferred_element_type=jnp.float32)
          + b7_ref[...] + ac_ref[...] * w8_ref[...] + b8_ref[...]
          + sc_ref[...] * w9_ref[...] + b9_ref[...])
    hh = jnp.maximum(jnp.concatenate([h1, h2], axis=1), 0.0)
    q_ref[...] = jnp.dot(hh, w5_ref[...],
                         preferred_element_type=jnp.float32) + b5_ref[...]


def _run_k2b(pooled, cur, state_c, action,
             W5, b5, W6, b6, W7, b7, W8, b8, W9, b9):
    full = pl.BlockSpec(memory_space=pltpu.MemorySpace.VMEM)
    return pl.pallas_call(
        _k2b_body,
        in_specs=[full] * 14,
        out_specs=full,
        out_shape=jax.ShapeDtypeStruct((B, 1), jnp.float32),
    )(pooled, cur, state_c, action,
      W5, b5, W6, b6, W7, b7, W8, b8, W9, b9)


def kernel(x, edge_index, edge_w, state_v, state_c, action,
           W1x, W1w, W1f, b1f, wvec1,
           W2x, W2w, W2f, b2f, wvec2,
           W5, b5, W6, b6, W7, b7, W8, b8, W9, b9):
    w = edge_w.reshape(E)
    xtab = jnp.pad(x, ((0, 0), (0, 14)))

    zeros_a = jnp.zeros((ZA, 16), jnp.float32)
    zeros_b = jnp.zeros((ZA, HF), jnp.float32)

    outa = _pass_a(edge_index, w, xtab, zeros_a)

    r1 = lambda v: v.reshape(1, -1)
    h = _run_k1(outa, x, W1x, W1f, r1(b1f), r1(wvec1), W1w)

    outb = _pass_b(edge_index, h, zeros_b)

    sv = state_v.reshape(B, 1)
    pooled, cur = _run_k2a(sv, x, outa, outb,
                           W2x, W2f, r1(b2f), r1(wvec2), W2w)
    return _run_k2b(pooled, cur, state_c, action,
                    W5, b5.reshape(1, 1), W6, r1(b6), W7, r1(b7),
                    r1(W8[0]), r1(b8), r1(W9[0]), r1(b9))
